# baseline (speedup 1.0000x reference)
import functools
import math

import jax
import jax.numpy as jnp
from jax import lax
from jax.experimental import pallas as pl
from jax.experimental.pallas import tpu as pltpu
from jax.experimental.pallas import tpu_sc as plsc

D = 128
SCALE = math.sqrt(128.0)
LANES = 16
NIN = 3
NOUT = 2
UNROLL = 12


def _sc_embed(idx3, lut, n_chunks, chunk, b_per_w):
    mesh = plsc.VectorSubcoreMesh(core_axis_name="c", subcore_axis_name="s")
    info = plsc.get_sparse_core_info()
    nc = info.num_cores
    B = idx3.shape[0] * idx3.shape[1] * idx3.shape[2]
    n_main = (n_chunks // UNROLL) * UNROLL

    @functools.partial(
        pl.kernel,
        mesh=mesh,
        out_type=jax.ShapeDtypeStruct((B, D), jnp.float32),
        scratch_types=[
            pltpu.VMEM((n_chunks, chunk), jnp.int32),
            pltpu.VMEM((chunk, D), jnp.float32),
            pltpu.VMEM((chunk, D), jnp.float32),
            pltpu.VMEM((chunk, D), jnp.float32),
            pltpu.VMEM((2 * chunk, D), jnp.float32),
            pltpu.VMEM((2 * chunk, D), jnp.float32),
            pltpu.SemaphoreType.DMA,
            pltpu.SemaphoreType.DMA,
            pltpu.SemaphoreType.DMA,
            pltpu.SemaphoreType.DMA,
            pltpu.SemaphoreType.DMA,
        ],
    )
    def k(idx_hbm, lut_hbm, out_hbm, idx_v,
          in0, in1, in2, ot0, ot1,
          gs0, gs1, gs2, os0, os1):
        wid = lax.axis_index("s") * nc + lax.axis_index("c")
        pltpu.sync_copy(idx_hbm.at[wid], idx_v)
        base = wid * b_per_w
        ins = (in0, in1, in2)
        outs = (ot0, ot1)
        gsems = (gs0, gs1, gs2)
        osems = (os0, os1)

        for b in range(NIN):
            pltpu.async_copy(lut_hbm.at[idx_v.at[b]], ins[b], gsems[b])

        def step(c, b, ob, half):
            pltpu.make_async_copy(lut_hbm.at[idx_v.at[c]], ins[b], gsems[b]).wait()

            if half == 0:
                @pl.when(c >= 2 * NOUT)
                def _():
                    pltpu.make_async_copy(
                        outs[ob], out_hbm.at[pl.ds(base, 2 * chunk)], osems[ob]
                    ).wait()

            def row_body(r, carry2):
                for j in range(D // LANES):
                    sl = pl.ds(j * LANES, LANES)
                    outs[ob][half * chunk + r, sl] = ins[b][r, sl] * SCALE
                return carry2

            lax.fori_loop(0, chunk, row_body, 0)

            @pl.when(c + NIN < n_chunks)
            def _():
                pltpu.async_copy(lut_hbm.at[idx_v.at[c + NIN]], ins[b], gsems[b])

            if half == 1:
                pltpu.async_copy(
                    outs[ob],
                    out_hbm.at[pl.ds(base + (c - 1) * chunk, 2 * chunk)],
                    osems[ob],
                )

        def outer(g, carry):
            for kk in range(UNROLL):
                c = g * UNROLL + kk
                step(c, kk % NIN, (kk // 2) % NOUT, kk % 2)
            return carry

        lax.fori_loop(0, n_main // UNROLL, outer, 0)

        for cc in range(n_main, n_chunks):
            step(cc, cc % NIN, ((cc % UNROLL) // 2) % NOUT, cc % 2)

        for ob in range(NOUT):
            pltpu.make_async_copy(
                outs[ob], out_hbm.at[pl.ds(base, 2 * chunk)], osems[ob]
            ).wait()

    return k(idx3, lut)


def kernel(x, lut):
    n_rows, n_cols = x.shape
    B = n_rows * n_cols
    nw = 32
    chunk = 128
    b_per_w = B // nw
    n_chunks = b_per_w // chunk
    idx3 = x.T.reshape(nw, n_chunks, chunk).astype(jnp.int32)
    out = _sc_embed(idx3, lut, n_chunks, chunk, b_per_w)
    return out.reshape(n_cols, n_rows, D).transpose(1, 0, 2)


# speedup vs baseline: 1.0080x; 1.0080x over previous
import functools
import math

import jax
import jax.numpy as jnp
from jax import lax
from jax.experimental import pallas as pl
from jax.experimental.pallas import tpu as pltpu
from jax.experimental.pallas import tpu_sc as plsc

D = 128
SCALE = math.sqrt(128.0)
LANES = 16
NBUF = 3


def _sc_embed(idx3, lut, n_chunks, chunk, b_per_w):
    mesh = plsc.VectorSubcoreMesh(core_axis_name="c", subcore_axis_name="s")
    info = plsc.get_sparse_core_info()
    nc = info.num_cores
    B = idx3.shape[0] * idx3.shape[1] * idx3.shape[2]
    n_main = (n_chunks // NBUF) * NBUF

    @functools.partial(
        pl.kernel,
        mesh=mesh,
        out_type=jax.ShapeDtypeStruct((B, D), jnp.float32),
        scratch_types=[
            pltpu.VMEM((n_chunks, chunk), jnp.int32),
            pltpu.VMEM((chunk, D), jnp.float32),
            pltpu.VMEM((chunk, D), jnp.float32),
            pltpu.VMEM((chunk, D), jnp.float32),
            pltpu.VMEM((chunk, D), jnp.float32),
            pltpu.VMEM((chunk, D), jnp.float32),
            pltpu.VMEM((chunk, D), jnp.float32),
            pltpu.SemaphoreType.DMA,
            pltpu.SemaphoreType.DMA,
            pltpu.SemaphoreType.DMA,
            pltpu.SemaphoreType.DMA,
            pltpu.SemaphoreType.DMA,
            pltpu.SemaphoreType.DMA,
        ],
    )
    def k(idx_hbm, lut_hbm, out_hbm, idx_v,
          in0, in1, in2, ot0, ot1, ot2,
          gs0, gs1, gs2, os0, os1, os2):
        wid = lax.axis_index("s") * nc + lax.axis_index("c")
        pltpu.sync_copy(idx_hbm.at[wid], idx_v)
        base = wid * b_per_w
        ins = (in0, in1, in2)
        outs = (ot0, ot1, ot2)
        gsems = (gs0, gs1, gs2)
        osems = (os0, os1, os2)

        for b in range(NBUF):
            pltpu.async_copy(lut_hbm.at[idx_v.at[b]], ins[b], gsems[b])

        def step(c, b):
            pltpu.make_async_copy(lut_hbm.at[idx_v.at[c]], ins[b], gsems[b]).wait()

            @pl.when(c >= NBUF)
            def _():
                pltpu.make_async_copy(
                    outs[b], out_hbm.at[pl.ds(base, chunk)], osems[b]
                ).wait()

            def row_body(r, carry2):
                for j in range(D // LANES):
                    sl = pl.ds(j * LANES, LANES)
                    outs[b][r, sl] = ins[b][r, sl] * SCALE
                return carry2

            lax.fori_loop(0, chunk, row_body, 0)

            @pl.when(c + NBUF < n_chunks)
            def _():
                pltpu.async_copy(lut_hbm.at[idx_v.at[c + NBUF]], ins[b], gsems[b])

            pltpu.async_copy(
                outs[b], out_hbm.at[pl.ds(base + c * chunk, chunk)], osems[b]
            )

        def outer(g, carry):
            for b in range(NBUF):
                step(g * NBUF + b, b)
            return carry

        lax.fori_loop(0, n_main // NBUF, outer, 0)

        for c in range(n_main, n_chunks):
            step(c, c % NBUF)

        for b in range(NBUF):
            pltpu.make_async_copy(
                outs[b], out_hbm.at[pl.ds(base, chunk)], osems[b]
            ).wait()

    return k(idx3, lut)


def kernel(x, lut):
    n_rows, n_cols = x.shape
    B = n_rows * n_cols
    nw = 32
    chunk = 128
    b_per_w = B // nw
    n_chunks = b_per_w // chunk
    idx3 = x.T.reshape(nw, n_chunks, chunk).astype(jnp.int32)
    out = _sc_embed(idx3, lut, n_chunks, chunk, b_per_w)
    return out.reshape(n_cols, n_rows, D).transpose(1, 0, 2)
